# baseline (speedup 1.0000x reference)
import jax
import jax.numpy as jnp
from jax import lax
from jax.experimental import pallas as pl
from jax.experimental.pallas import tpu as pltpu
from jax.experimental.pallas import tpu_sc as plsc

_N, _M, _D = 10000, 16, 256
_ROW = _M * _D
_L = 16
_NW = 32
_C = 8

_BASE_CNT = _N // _NW
_EXTRA = _N - _BASE_CNT * _NW


def _sc_body(mail_hbm, out_hbm, buf, obuf, isem, osem):
    cid = lax.axis_index("c")
    sid = lax.axis_index("s")
    wid = sid * 2 + cid
    w_start = wid * _BASE_CNT + jnp.minimum(wid, _EXTRA)
    w_cnt = _BASE_CNT + (wid < _EXTRA).astype(jnp.int32)
    nblk = (w_cnt + _C - 1) // _C

    def blk_nodes(i):
        return w_start + jnp.minimum(i * _C, w_cnt - _C)

    def loop_body(i, carry):
        n0 = blk_nodes(i)
        pltpu.sync_copy(
            mail_hbm.at[pl.ds(n0 * _ROW, _C * _ROW)],
            buf.at[pl.ds(0, _C * _ROW)],
        )

        def node_body(j, carry2):
            boff = j * _ROW
            ooff = j * _D
            for f in range(_D // _L):
                acc = buf[pl.ds(boff + f * _L, _L)]
                for m in range(1, _M):
                    acc = jnp.maximum(acc, buf[pl.ds(boff + m * _D + f * _L, _L)])
                obuf[pl.ds(ooff + f * _L, _L)] = acc
            return carry2

        lax.fori_loop(0, _C, node_body, 0)
        pltpu.sync_copy(
            obuf.at[pl.ds(0, _C * _D)],
            out_hbm.at[pl.ds(n0 * _D, _C * _D)],
        )
        return carry

    lax.fori_loop(0, nblk, loop_body, 0)


def _sc_call(mail_flat):
    mesh = plsc.VectorSubcoreMesh(core_axis_name="c", subcore_axis_name="s")
    return pl.kernel(
        _sc_body,
        out_type=jax.ShapeDtypeStruct((_N * _D,), jnp.float32),
        mesh=mesh,
        scratch_types=[
            pltpu.VMEM((_C * _ROW,), jnp.float32),
            pltpu.VMEM((_C * _D,), jnp.float32),
            pltpu.SemaphoreType.DMA,
            pltpu.SemaphoreType.DMA,
        ],
    )(mail_flat)


def kernel(mailbox):
    n, m, d = mailbox.shape
    out_flat = _sc_call(mailbox.reshape(-1))
    return out_flat.reshape(n, d)


# speedup vs baseline: 1.3078x; 1.3078x over previous
import jax
import jax.numpy as jnp
from jax import lax
from jax.experimental import pallas as pl
from jax.experimental.pallas import tpu as pltpu
from jax.experimental.pallas import tpu_sc as plsc

_N, _M, _D = 10000, 16, 256
_ROW = _M * _D
_L = 16
_NW = 32
_C = 8
_W_CNT = 313
_NBLK = (_W_CNT + _C - 1) // _C
_PAIRS = _NBLK // 2


def _sc_body(mail_hbm, out_hbm, buf, obuf, isem0, isem1, osem0, osem1):
    cid = lax.axis_index("c")
    sid = lax.axis_index("s")
    wid = sid * 2 + cid
    w_start = jnp.minimum(wid * _W_CNT, _N - _W_CNT)

    def blk_node0(i):
        return w_start + jnp.minimum(i * _C, _W_CNT - _C)

    def start_in(i, slot, sem):
        n0 = blk_node0(i)
        pltpu.async_copy(
            mail_hbm.at[pl.ds(n0 * _ROW, _C * _ROW)],
            buf.at[pl.ds(slot * (_C * _ROW), _C * _ROW)],
            sem,
        )

    def wait_in(slot, sem):
        pltpu.make_async_copy(
            mail_hbm.at[pl.ds(0, _C * _ROW)],
            buf.at[pl.ds(slot * (_C * _ROW), _C * _ROW)],
            sem,
        ).wait()

    def start_out(i, slot, sem):
        n0 = blk_node0(i)
        pltpu.async_copy(
            obuf.at[pl.ds(slot * (_C * _D), _C * _D)],
            out_hbm.at[pl.ds(n0 * _D, _C * _D)],
            sem,
        )

    def wait_out(slot, sem):
        pltpu.make_async_copy(
            obuf.at[pl.ds(slot * (_C * _D), _C * _D)],
            out_hbm.at[pl.ds(0, _C * _D)],
            sem,
        ).wait()

    def compute_block(slot):
        base = slot * (_C * _ROW)
        obase = slot * (_C * _D)

        def node_body(j, carry):
            boff = base + j * _ROW
            ooff = obase + j * _D
            for f in range(_D // _L):
                acc = buf[pl.ds(boff + f * _L, _L)]
                for m in range(1, _M):
                    acc = jnp.maximum(acc, buf[pl.ds(boff + m * _D + f * _L, _L)])
                obuf[pl.ds(ooff + f * _L, _L)] = acc
            return carry

        lax.fori_loop(0, _C, node_body, 0)

    start_in(0, 0, isem0)

    def pair_body(p, carry):
        i0 = 2 * p
        i1 = i0 + 1
        start_in(i1, 1, isem1)
        wait_in(0, isem0)

        @pl.when(p >= 1)
        def _():
            wait_out(0, osem0)

        compute_block(0)
        start_out(i0, 0, osem0)

        @pl.when(p < _PAIRS - 1)
        def _():
            start_in(i0 + 2, 0, isem0)

        wait_in(1, isem1)

        @pl.when(p >= 1)
        def _():
            wait_out(1, osem1)

        compute_block(1)
        start_out(i1, 1, osem1)
        return carry

    lax.fori_loop(0, _PAIRS, pair_body, 0)
    wait_out(0, osem0)
    wait_out(1, osem1)


def _sc_call(mail_flat):
    mesh = plsc.VectorSubcoreMesh(core_axis_name="c", subcore_axis_name="s")
    return pl.kernel(
        _sc_body,
        out_type=jax.ShapeDtypeStruct((_N * _D,), jnp.float32),
        mesh=mesh,
        scratch_types=[
            pltpu.VMEM((2 * _C * _ROW,), jnp.float32),
            pltpu.VMEM((2 * _C * _D,), jnp.float32),
            pltpu.SemaphoreType.DMA,
            pltpu.SemaphoreType.DMA,
            pltpu.SemaphoreType.DMA,
            pltpu.SemaphoreType.DMA,
        ],
    )(mail_flat)


def kernel(mailbox):
    n, m, d = mailbox.shape
    out_flat = _sc_call(mailbox.reshape(-1))
    return out_flat.reshape(n, d)
